# baseline (speedup 1.0000x reference)
import functools

import jax
import jax.numpy as jnp
from jax import lax
from jax.experimental import pallas as pl
from jax.experimental.pallas import tpu as pltpu
from jax.experimental.pallas import tpu_sc as plsc

NC, NS = 2, 16
NW = NC * NS
B = 4096
P = B // 2
K = 20
D = 64
D2 = 2 * D
NR = 16
CHUNK = 128

ROWS_PER_W = (B * K) // NW
N_CHUNKS = ROWS_PER_W // CHUNK
PAIRS_PER_W = ROWS_PER_W // 2


def _wid():
    return lax.axis_index("s") * NC + lax.axis_index("c")


def _sc_gather_t_body(ent_hbm, idx_hbm, ih_hbm, t_out, h_out,
                      idx_v, hidx_v, hbuf, buf0, buf1, sem0, sem1, hsem):
    wid = _wid()
    base = wid * ROWS_PER_W
    pbase = wid * PAIRS_PER_W
    hbase = wid * CHUNK
    pltpu.sync_copy(ih_hbm.at[pl.ds(hbase, CHUNK)], hidx_v)
    hc = pltpu.async_copy(ent_hbm.at[hidx_v], hbuf, hsem)
    pltpu.sync_copy(idx_hbm.at[pl.ds(base, ROWS_PER_W)], idx_v)
    bufs = (buf0, buf1)
    sems = (sem0, sem1)
    H = CHUNK // 2

    def _write(c, wait_only):
        bf = bufs[c % 2]
        prow = pbase + c * H
        for half in (0, 1):
            cp = pltpu.make_async_copy(
                bf.at[pl.ds(half * H, H)],
                t_out.at[pl.ds(prow, H), pl.ds(half * D, D)], sems[c % 2])
            if wait_only:
                cp.wait()
            else:
                cp.start()

    cps = [None, None]
    cps[0] = pltpu.async_copy(ent_hbm.at[idx_v.at[pl.ds(0, CHUNK)]], buf0, sem0)
    for c in range(N_CHUNKS):
        nxt = c + 1
        if nxt < N_CHUNKS:
            if nxt >= 2:
                _write(nxt - 2, wait_only=True)
            cps[nxt % 2] = pltpu.async_copy(
                ent_hbm.at[idx_v.at[pl.ds(nxt * CHUNK, CHUNK)]],
                bufs[nxt % 2], sems[nxt % 2])
        cps[c % 2].wait()
        _write(c, wait_only=False)
    _write(N_CHUNKS - 2, wait_only=True)
    _write(N_CHUNKS - 1, wait_only=True)
    hc.wait()
    hprow = wid * (CHUNK // 2)
    pltpu.sync_copy(hbuf.at[pl.ds(0, CHUNK // 2)],
                    h_out.at[pl.ds(hprow, CHUNK // 2), pl.ds(0, D)])
    pltpu.sync_copy(hbuf.at[pl.ds(CHUNK // 2, CHUNK // 2)],
                    h_out.at[pl.ds(hprow, CHUNK // 2), pl.ds(D, D)])


@functools.lru_cache(maxsize=None)
def _sc_kernels():
    mesh = plsc.VectorSubcoreMesh(
        core_axis_name="c", subcore_axis_name="s",
        num_cores=NC, num_subcores=NS)
    cp = pltpu.CompilerParams(use_tc_tiling_on_sc=False)
    gather_t = pl.kernel(
        _sc_gather_t_body,
        compiler_params=cp,
        out_type=(
            jax.ShapeDtypeStruct((B * K // 2, D2), jnp.float32),
            jax.ShapeDtypeStruct((P, D2), jnp.float32),
        ),
        mesh=mesh,
        scratch_types=[
            pltpu.VMEM((ROWS_PER_W,), jnp.int32),
            pltpu.VMEM((CHUNK,), jnp.int32),
            pltpu.VMEM((CHUNK, D), jnp.float32),
            pltpu.VMEM((CHUNK, D), jnp.float32),
            pltpu.VMEM((CHUNK, D), jnp.float32),
            pltpu.SemaphoreType.DMA,
            pltpu.SemaphoreType.DMA,
            pltpu.SemaphoreType.DMA,
        ],
    )
    return gather_t


PB = 2048
N_BLOCKS = P // PB


def _renorm(x):
    return x * lax.rsqrt(jnp.maximum(jnp.sum(x * x, axis=1, keepdims=True),
                                     1.0))


def _seg_mats():
    r = lax.broadcasted_iota(jnp.int32, (D2, 2), 0)
    c = lax.broadcasted_iota(jnp.int32, (D2, 2), 1)
    s2 = jnp.where((r < D) == (c == 0), 1.0, 0.0)
    rb = lax.broadcasted_iota(jnp.int32, (2, D2), 0)
    cb = lax.broadcasted_iota(jnp.int32, (2, D2), 1)
    sb = jnp.where((cb < D) == (rb == 0), 1.0, 0.0)
    r4 = lax.broadcasted_iota(jnp.int32, (2, 2 * K), 0)
    c4 = lax.broadcasted_iota(jnp.int32, (2, 2 * K), 1)
    sb40 = jnp.where((c4 < K) == (r4 == 0), 1.0, 0.0)
    return s2, sb, sb40


def _dense_body(u_ref, h_ref, t_ref, rid_ref, rel_ref, w1_ref, b1_ref,
                w2_ref, b2_ref, out_ref):
    s2m, sbm, sb40m = _seg_mats()

    def seg_sum(x):
        return lax.dot_general(x, s2m, (((1,), (0,)), ((), ())),
                               preferred_element_type=jnp.float32)

    def seg_bcast(v):
        return lax.dot_general(v, sbm, (((1,), (0,)), ((), ())),
                               preferred_element_type=jnp.float32)

    def seg40_bcast(v):
        return lax.dot_general(v, sb40m, (((1,), (0,)), ((), ())),
                               preferred_element_type=jnp.float32)

    def renorm_pack(x):
        return x * seg_bcast(lax.rsqrt(jnp.maximum(seg_sum(x * x), 1.0)))

    u = renorm_pack(u_ref[...])
    h = renorm_pack(h_ref[...])
    rel = _renorm(rel_ref[...])
    dn = (((1,), (1,)), ((), ()))
    se = lax.dot_general(u[:, :D], rel, dn,
                         preferred_element_type=jnp.float32)
    so = lax.dot_general(u[:, D:], rel, dn,
                         preferred_element_type=jnp.float32)
    rid = rid_ref[...]
    ur = jnp.zeros((PB, 2 * K), jnp.float32)
    for j in range(NR):
        bj = seg40_bcast(jnp.concatenate([se[:, j:j + 1], so[:, j:j + 1]], 1))
        ur = jnp.where(rid == j, bj, ur)
    m_e = jnp.max(ur[:, :K], axis=1, keepdims=True)
    m_o = jnp.max(ur[:, K:], axis=1, keepdims=True)
    ex = jnp.exp(ur - seg40_bcast(jnp.concatenate([m_e, m_o], 1)))
    sm_e = jnp.sum(ex[:, :K], axis=1, keepdims=True)
    sm_o = jnp.sum(ex[:, K:], axis=1, keepdims=True)
    w = ex * seg40_bcast(1.0 / jnp.concatenate([sm_e, sm_o], 1))
    s = jnp.zeros((PB, D2), jnp.float32)
    ss = jnp.zeros((PB, D2), jnp.float32)
    for k in range(K):
        x = t_ref[k]
        scl = lax.rsqrt(jnp.maximum(seg_sum(x * x), 1.0))
        wk = jnp.concatenate([w[:, k:k + 1], w[:, K + k:K + k + 1]], 1)
        xt = x * seg_bcast(scl * wk)
        s = s + xt
        ss = ss + xt * xt
    nh = s * s - ss
    z1 = h + nh
    z2 = h * nh
    x1 = jnp.concatenate(
        [lax.dot_general(z1[:, :D], w1_ref[...], dn,
                         preferred_element_type=jnp.float32),
         lax.dot_general(z1[:, D:], w1_ref[...], dn,
                         preferred_element_type=jnp.float32)], 1) + b1_ref[...]
    x2 = jnp.concatenate(
        [lax.dot_general(z2[:, :D], w2_ref[...], dn,
                         preferred_element_type=jnp.float32),
         lax.dot_general(z2[:, D:], w2_ref[...], dn,
                         preferred_element_type=jnp.float32)], 1) + b2_ref[...]
    item = jnp.where(x1 >= 0, x1, 0.2 * x1) + jnp.where(x2 >= 0, x2, 0.2 * x2)
    logit = seg_sum(u * item)
    out_ref[...] = 1.0 / (1.0 + jnp.exp(-logit))


def _dense_call(u_pack, h_pack, t_kmaj, r_pack, rel, w1, b1, w2, b2,
                interpret=False):
    return pl.pallas_call(
        _dense_body,
        grid=(N_BLOCKS,),
        in_specs=[
            pl.BlockSpec((PB, D2), lambda b: (b, 0)),
            pl.BlockSpec((PB, D2), lambda b: (b, 0)),
            pl.BlockSpec((K, PB, D2), lambda b: (0, b, 0)),
            pl.BlockSpec((PB, 2 * K), lambda b: (b, 0)),
            pl.BlockSpec((NR, D), lambda b: (0, 0)),
            pl.BlockSpec((D, D), lambda b: (0, 0)),
            pl.BlockSpec((1, D2), lambda b: (0, 0)),
            pl.BlockSpec((D, D), lambda b: (0, 0)),
            pl.BlockSpec((1, D2), lambda b: (0, 0)),
        ],
        out_specs=pl.BlockSpec((PB, 2), lambda b: (b, 0)),
        out_shape=jax.ShapeDtypeStruct((P, 2), jnp.float32),
        interpret=interpret,
    )(u_pack, h_pack, t_kmaj, r_pack, rel, w1, b1, w2, b2)


def _pair_order(v):
    ev = v[0::2].reshape(-1, 64)
    od = v[1::2].reshape(-1, 64)
    return jnp.concatenate([ev, od], axis=1).reshape(-1)


def kernel(u, i, adj_entity, adj_relation, user_table, entity_table,
           relation_table, W1_w, W1_b, W2_w, W2_b):
    i = i.astype(jnp.int32)
    u = u.astype(jnp.int32)
    gather_t = _sc_kernels()
    e_ids = jnp.take(adj_entity, i, axis=0, mode="clip").astype(jnp.int32)
    r_ids = jnp.take(adj_relation, i, axis=0, mode="clip").astype(jnp.int32)
    u_raw = jnp.take(user_table, u, axis=0, mode="clip")
    e_paired = _pair_order(e_ids.T.reshape(-1))
    i_paired = _pair_order(i)
    t_pack, h_pack = gather_t(entity_table, e_paired, i_paired)
    t_kmaj = t_pack.reshape(K, P, D2)
    u_pack = u_raw.reshape(P, D2)
    r_pack = r_ids.reshape(P, 2 * K)
    b1_2 = jnp.concatenate([W1_b, W1_b]).reshape(1, D2)
    b2_2 = jnp.concatenate([W2_b, W2_b]).reshape(1, D2)
    out = _dense_call(u_pack, h_pack, t_kmaj, r_pack, relation_table,
                      W1_w, b1_2, W2_w, b2_2)
    return out.reshape(B)


# speedup vs baseline: 1.0095x; 1.0095x over previous
import functools

import jax
import jax.numpy as jnp
from jax import lax
from jax.experimental import pallas as pl
from jax.experimental.pallas import tpu as pltpu
from jax.experimental.pallas import tpu_sc as plsc

NC, NS = 2, 16
NW = NC * NS
B = 4096
P = B // 2
K = 20
D = 64
D2 = 2 * D
NR = 16
CHUNK = 128

ROWS_PER_W = (B * K) // NW
N_CHUNKS = ROWS_PER_W // CHUNK
PAIRS_PER_W = ROWS_PER_W // 2


def _wid():
    return lax.axis_index("s") * NC + lax.axis_index("c")


def _sc_gather_t_body(ent_hbm, idx_hbm, ih_hbm, t_out, h_out,
                      idx_v, hidx_v, hbuf, buf0, buf1, sem0, sem1, hsem):
    wid = _wid()
    base = wid * ROWS_PER_W
    pbase = wid * PAIRS_PER_W
    hbase = wid * CHUNK
    pltpu.sync_copy(ih_hbm.at[pl.ds(hbase, CHUNK)], hidx_v)
    hc = pltpu.async_copy(ent_hbm.at[hidx_v], hbuf, hsem)
    pltpu.sync_copy(idx_hbm.at[pl.ds(base, ROWS_PER_W)], idx_v)
    bufs = (buf0, buf1)
    sems = (sem0, sem1)
    H = CHUNK // 2

    def _write(c, wait_only):
        bf = bufs[c % 2]
        prow = pbase + c * H
        for half in (0, 1):
            cp = pltpu.make_async_copy(
                bf.at[pl.ds(half * H, H)],
                t_out.at[pl.ds(prow, H), pl.ds(half * D, D)], sems[c % 2])
            if wait_only:
                cp.wait()
            else:
                cp.start()

    cps = [None, None]
    cps[0] = pltpu.async_copy(ent_hbm.at[idx_v.at[pl.ds(0, CHUNK)]], buf0, sem0)
    for c in range(N_CHUNKS):
        nxt = c + 1
        if nxt < N_CHUNKS:
            if nxt >= 2:
                _write(nxt - 2, wait_only=True)
            cps[nxt % 2] = pltpu.async_copy(
                ent_hbm.at[idx_v.at[pl.ds(nxt * CHUNK, CHUNK)]],
                bufs[nxt % 2], sems[nxt % 2])
        cps[c % 2].wait()
        _write(c, wait_only=False)
    _write(N_CHUNKS - 2, wait_only=True)
    _write(N_CHUNKS - 1, wait_only=True)
    hc.wait()
    hprow = wid * (CHUNK // 2)
    pltpu.sync_copy(hbuf.at[pl.ds(0, CHUNK // 2)],
                    h_out.at[pl.ds(hprow, CHUNK // 2), pl.ds(0, D)])
    pltpu.sync_copy(hbuf.at[pl.ds(CHUNK // 2, CHUNK // 2)],
                    h_out.at[pl.ds(hprow, CHUNK // 2), pl.ds(D, D)])


@functools.lru_cache(maxsize=None)
def _sc_kernels():
    mesh = plsc.VectorSubcoreMesh(
        core_axis_name="c", subcore_axis_name="s",
        num_cores=NC, num_subcores=NS)
    cp = pltpu.CompilerParams(use_tc_tiling_on_sc=False)
    gather_t = pl.kernel(
        _sc_gather_t_body,
        compiler_params=cp,
        out_type=(
            jax.ShapeDtypeStruct((B * K // 2, D2), jnp.float32),
            jax.ShapeDtypeStruct((P, D2), jnp.float32),
        ),
        mesh=mesh,
        scratch_types=[
            pltpu.VMEM((ROWS_PER_W,), jnp.int32),
            pltpu.VMEM((CHUNK,), jnp.int32),
            pltpu.VMEM((CHUNK, D), jnp.float32),
            pltpu.VMEM((CHUNK, D), jnp.float32),
            pltpu.VMEM((CHUNK, D), jnp.float32),
            pltpu.SemaphoreType.DMA,
            pltpu.SemaphoreType.DMA,
            pltpu.SemaphoreType.DMA,
        ],
    )
    return gather_t


PB = 1024
N_BLOCKS = P // PB


def _renorm(x):
    return x * lax.rsqrt(jnp.maximum(jnp.sum(x * x, axis=1, keepdims=True),
                                     1.0))


def _seg_mats():
    r = lax.broadcasted_iota(jnp.int32, (D2, 2), 0)
    c = lax.broadcasted_iota(jnp.int32, (D2, 2), 1)
    s2 = jnp.where((r < D) == (c == 0), 1.0, 0.0)
    rb = lax.broadcasted_iota(jnp.int32, (2, D2), 0)
    cb = lax.broadcasted_iota(jnp.int32, (2, D2), 1)
    sb = jnp.where((cb < D) == (rb == 0), 1.0, 0.0)
    r4 = lax.broadcasted_iota(jnp.int32, (2, 2 * K), 0)
    c4 = lax.broadcasted_iota(jnp.int32, (2, 2 * K), 1)
    sb40 = jnp.where((c4 < K) == (r4 == 0), 1.0, 0.0)
    return s2, sb, sb40


def _dense_body(u_ref, h_ref, t_ref, rid_ref, rel_ref, w1_ref, b1_ref,
                w2_ref, b2_ref, out_ref):
    s2m, sbm, sb40m = _seg_mats()

    def seg_sum(x):
        return lax.dot_general(x, s2m, (((1,), (0,)), ((), ())),
                               preferred_element_type=jnp.float32)

    def seg_bcast(v):
        return lax.dot_general(v, sbm, (((1,), (0,)), ((), ())),
                               preferred_element_type=jnp.float32)

    def seg40_bcast(v):
        return lax.dot_general(v, sb40m, (((1,), (0,)), ((), ())),
                               preferred_element_type=jnp.float32)

    def renorm_pack(x):
        return x * seg_bcast(lax.rsqrt(jnp.maximum(seg_sum(x * x), 1.0)))

    u = renorm_pack(u_ref[...])
    h = renorm_pack(h_ref[...])
    rel = _renorm(rel_ref[...])
    dn = (((1,), (1,)), ((), ()))
    se = lax.dot_general(u[:, :D], rel, dn,
                         preferred_element_type=jnp.float32)
    so = lax.dot_general(u[:, D:], rel, dn,
                         preferred_element_type=jnp.float32)
    rid = rid_ref[...]
    ur = jnp.zeros((PB, 2 * K), jnp.float32)
    for j in range(NR):
        bj = seg40_bcast(jnp.concatenate([se[:, j:j + 1], so[:, j:j + 1]], 1))
        ur = jnp.where(rid == j, bj, ur)
    m_e = jnp.max(ur[:, :K], axis=1, keepdims=True)
    m_o = jnp.max(ur[:, K:], axis=1, keepdims=True)
    ex = jnp.exp(ur - seg40_bcast(jnp.concatenate([m_e, m_o], 1)))
    sm_e = jnp.sum(ex[:, :K], axis=1, keepdims=True)
    sm_o = jnp.sum(ex[:, K:], axis=1, keepdims=True)
    w = ex * seg40_bcast(1.0 / jnp.concatenate([sm_e, sm_o], 1))
    s = jnp.zeros((PB, D2), jnp.float32)
    ss = jnp.zeros((PB, D2), jnp.float32)
    for k in range(K):
        x = t_ref[k]
        scl = lax.rsqrt(jnp.maximum(seg_sum(x * x), 1.0))
        wk = jnp.concatenate([w[:, k:k + 1], w[:, K + k:K + k + 1]], 1)
        xt = x * seg_bcast(scl * wk)
        s = s + xt
        ss = ss + xt * xt
    nh = s * s - ss
    z1 = h + nh
    z2 = h * nh
    x1 = jnp.concatenate(
        [lax.dot_general(z1[:, :D], w1_ref[...], dn,
                         preferred_element_type=jnp.float32),
         lax.dot_general(z1[:, D:], w1_ref[...], dn,
                         preferred_element_type=jnp.float32)], 1) + b1_ref[...]
    x2 = jnp.concatenate(
        [lax.dot_general(z2[:, :D], w2_ref[...], dn,
                         preferred_element_type=jnp.float32),
         lax.dot_general(z2[:, D:], w2_ref[...], dn,
                         preferred_element_type=jnp.float32)], 1) + b2_ref[...]
    item = jnp.where(x1 >= 0, x1, 0.2 * x1) + jnp.where(x2 >= 0, x2, 0.2 * x2)
    logit = seg_sum(u * item)
    out_ref[...] = 1.0 / (1.0 + jnp.exp(-logit))


def _dense_call(u_pack, h_pack, t_kmaj, r_pack, rel, w1, b1, w2, b2,
                interpret=False):
    return pl.pallas_call(
        _dense_body,
        grid=(N_BLOCKS,),
        in_specs=[
            pl.BlockSpec((PB, D2), lambda b: (b, 0)),
            pl.BlockSpec((PB, D2), lambda b: (b, 0)),
            pl.BlockSpec((K, PB, D2), lambda b: (0, b, 0)),
            pl.BlockSpec((PB, 2 * K), lambda b: (b, 0)),
            pl.BlockSpec((NR, D), lambda b: (0, 0)),
            pl.BlockSpec((D, D), lambda b: (0, 0)),
            pl.BlockSpec((1, D2), lambda b: (0, 0)),
            pl.BlockSpec((D, D), lambda b: (0, 0)),
            pl.BlockSpec((1, D2), lambda b: (0, 0)),
        ],
        out_specs=pl.BlockSpec((PB, 2), lambda b: (b, 0)),
        out_shape=jax.ShapeDtypeStruct((P, 2), jnp.float32),
        interpret=interpret,
    )(u_pack, h_pack, t_kmaj, r_pack, rel, w1, b1, w2, b2)


def _pair_order(v):
    ev = v[0::2].reshape(-1, 64)
    od = v[1::2].reshape(-1, 64)
    return jnp.concatenate([ev, od], axis=1).reshape(-1)


def kernel(u, i, adj_entity, adj_relation, user_table, entity_table,
           relation_table, W1_w, W1_b, W2_w, W2_b):
    i = i.astype(jnp.int32)
    u = u.astype(jnp.int32)
    gather_t = _sc_kernels()
    e_ids = jnp.take(adj_entity, i, axis=0, mode="clip").astype(jnp.int32)
    r_ids = jnp.take(adj_relation, i, axis=0, mode="clip").astype(jnp.int32)
    u_raw = jnp.take(user_table, u, axis=0, mode="clip")
    e_paired = _pair_order(e_ids.T.reshape(-1))
    i_paired = _pair_order(i)
    t_pack, h_pack = gather_t(entity_table, e_paired, i_paired)
    t_kmaj = t_pack.reshape(K, P, D2)
    u_pack = u_raw.reshape(P, D2)
    r_pack = r_ids.reshape(P, 2 * K)
    b1_2 = jnp.concatenate([W1_b, W1_b]).reshape(1, D2)
    b2_2 = jnp.concatenate([W2_b, W2_b]).reshape(1, D2)
    out = _dense_call(u_pack, h_pack, t_kmaj, r_pack, relation_table,
                      W1_w, b1_2, W2_w, b2_2)
    return out.reshape(B)
